# baseline (speedup 1.0000x reference)
import functools

import jax
import jax.numpy as jnp
from jax import lax
from jax.experimental import pallas as pl
from jax.experimental.pallas import tpu as pltpu
from jax.experimental.pallas import tpu_sc as plsc

_VOCAB = 1000000
_D = 64
_B = 16384
_W = 16
_TOT = _B * _W
_NC = 2
_NS = 16
_NW = _NC * _NS
_PER_W = _TOT // _NW
_CHUNK = 256
_NCH = _PER_W // _CHUNK
assert _NCH % 2 == 0

_mesh = plsc.VectorSubcoreMesh(core_axis_name="c", subcore_axis_name="s")


@functools.partial(
    pl.kernel,
    mesh=_mesh,
    out_type=jax.ShapeDtypeStruct((_TOT, _D), jnp.float32),
    compiler_params=pltpu.CompilerParams(use_tc_tiling_on_sc=True),
    scratch_types=[
        pltpu.VMEM((_NCH, _CHUNK), jnp.int32),
        pltpu.VMEM((2, _CHUNK, _D), jnp.float32),
        [pltpu.SemaphoreType.DMA] * 2,
        [pltpu.SemaphoreType.DMA] * 2,
    ],
)
def _gather_kernel(idx_hbm, table_hbm, out_hbm, idx_v, rows_v, gsems, ssems):
    wid = lax.axis_index("s") * _NC + lax.axis_index("c")
    base = wid * _PER_W
    pltpu.sync_copy(idx_hbm.at[wid], idx_v)

    def issue(c, slot):
        def grp_body(g, carry):
            vec = idx_v[c, pl.ds(g * 16, 16)]
            for i in range(16):
                pltpu.make_async_copy(
                    table_hbm.at[pl.ds(vec[i], 1)],
                    rows_v.at[slot, pl.ds(g * 16 + i, 1)],
                    gsems[slot],
                ).start()
            return carry

        lax.fori_loop(0, _CHUNK // 16, grp_body, 0)

    def drain(slot):
        pltpu.make_async_copy(
            table_hbm.at[pl.ds(0, _CHUNK)], rows_v.at[slot], gsems[slot]
        ).wait()

    def store_start(c, slot):
        pltpu.make_async_copy(
            rows_v.at[slot],
            out_hbm.at[pl.ds(base + c * _CHUNK, _CHUNK)],
            ssems[slot],
        ).start()

    def store_wait(slot):
        pltpu.make_async_copy(
            rows_v.at[slot], out_hbm.at[pl.ds(base, _CHUNK)], ssems[slot]
        ).wait()

    issue(0, 0)

    def pair_body(p, carry):
        c0 = 2 * p

        @pl.when(p >= 1)
        def _():
            store_wait(1)
        issue(c0 + 1, 1)
        drain(0)
        store_start(c0, 0)

        @pl.when(c0 + 2 < _NCH)
        def _():
            store_wait(0)
            issue(c0 + 2, 0)
        drain(1)
        store_start(c0 + 1, 1)
        return carry

    lax.fori_loop(0, _NCH // 2, pair_body, 0)
    store_wait(0)
    store_wait(1)


def kernel(sentence, word_embd):
    idx = sentence.astype(jnp.int32).reshape(_NW, _NCH, _CHUNK)
    out = _gather_kernel(idx, word_embd)
    return out.reshape(_B, _W, _D)


# speedup vs baseline: 1.3460x; 1.3460x over previous
import functools

import jax
import jax.numpy as jnp
from jax import lax
from jax.experimental import pallas as pl
from jax.experimental.pallas import tpu as pltpu
from jax.experimental.pallas import tpu_sc as plsc

_VOCAB = 1000000
_D = 64
_B = 16384
_W = 16
_TOT = _B * _W
_NC = 2
_NS = 16
_NW = _NC * _NS
_PER_W = _TOT // _NW
_CHUNK = 256
_NCH = _PER_W // _CHUNK
assert _NCH % 2 == 0

_mesh = plsc.VectorSubcoreMesh(core_axis_name="c", subcore_axis_name="s")


@functools.partial(
    pl.kernel,
    mesh=_mesh,
    out_type=jax.ShapeDtypeStruct((_TOT, _D), jnp.float32),
    compiler_params=pltpu.CompilerParams(use_tc_tiling_on_sc=True),
    scratch_types=[
        pltpu.VMEM((_NCH, _CHUNK), jnp.int32),
        pltpu.VMEM((2, _CHUNK, _D), jnp.float32),
        [pltpu.SemaphoreType.DMA] * 2,
        [pltpu.SemaphoreType.DMA] * 2,
    ],
)
def _gather_kernel(idx_hbm, table3_hbm, out_hbm, idx_v, rows_v, gsems, ssems):
    table_hbm = table3_hbm.at[0]
    wid = lax.axis_index("s") * _NC + lax.axis_index("c")
    base = wid * _PER_W
    pltpu.sync_copy(idx_hbm.at[wid], idx_v)

    def issue(c, slot):
        def grp_body(g, carry):
            vec = idx_v[c, pl.ds(g * 16, 16)]
            for i in range(16):
                pltpu.make_async_copy(
                    table_hbm.at[pl.ds(vec[i], 1)],
                    rows_v.at[slot, pl.ds(g * 16 + i, 1)],
                    gsems[slot],
                ).start()
            return carry

        lax.fori_loop(0, _CHUNK // 16, grp_body, 0)

    def drain(slot):
        pltpu.make_async_copy(
            table_hbm.at[pl.ds(0, _CHUNK)], rows_v.at[slot], gsems[slot]
        ).wait()

    def store_start(c, slot):
        pltpu.make_async_copy(
            rows_v.at[slot],
            out_hbm.at[pl.ds(base + c * _CHUNK, _CHUNK)],
            ssems[slot],
        ).start()

    def store_wait(slot):
        pltpu.make_async_copy(
            rows_v.at[slot], out_hbm.at[pl.ds(base, _CHUNK)], ssems[slot]
        ).wait()

    issue(0, 0)

    def pair_body(p, carry):
        c0 = 2 * p

        @pl.when(p >= 1)
        def _():
            store_wait(1)
        issue(c0 + 1, 1)
        drain(0)
        store_start(c0, 0)

        @pl.when(c0 + 2 < _NCH)
        def _():
            store_wait(0)
            issue(c0 + 2, 0)
        drain(1)
        store_start(c0 + 1, 1)
        return carry

    lax.fori_loop(0, _NCH // 2, pair_body, 0)
    store_wait(0)
    store_wait(1)


def kernel(sentence, word_embd):
    idx = sentence.astype(jnp.int32).reshape(_NW, _NCH, _CHUNK)
    out = _gather_kernel(idx, word_embd.reshape(1, _VOCAB, _D))
    return out.reshape(_B, _W, _D)
